# baseline (speedup 1.0000x reference)
import functools

import jax
import jax.numpy as jnp
from jax import lax
from jax.experimental import pallas as pl
from jax.experimental.pallas import tpu as pltpu
from jax.experimental.pallas import tpu_sc as plsc

VOCAB = 100000
EMBED_DIM = 32
BATCH = 16384
H1, H2, H3 = 128, 64, 32
TPAD_ROWS = 100016


_PADBLK = 2048


def _padt_body(xt_ref, e_ref, o_ref):
    e = e_ref[...]
    o_ref[:, EMBED_DIM:] = jnp.zeros(
        (_PADBLK, 128 - EMBED_DIM), dtype=jnp.bfloat16
    )
    for t in range(_PADBLK // 128):
        tile = xt_ref[:, pl.ds(t * 128, 128)]
        tt = lax.dot_general(
            e, tile, (((1,), (1,)), ((), ())),
            preferred_element_type=jnp.float32,
        )
        o_ref[pl.ds(t * 128, 128), 0:EMBED_DIM] = tt.astype(jnp.bfloat16)


def _tc_pad(tT):
    nblk = (TPAD_ROWS + _PADBLK - 1) // _PADBLK
    eye = jnp.eye(128, dtype=jnp.float32)
    return pl.pallas_call(
        _padt_body,
        grid=(nblk,),
        in_specs=[
            pl.BlockSpec((EMBED_DIM, _PADBLK), lambda i: (0, i)),
            pl.BlockSpec((128, 128), lambda i: (0, 0)),
        ],
        out_specs=pl.BlockSpec((_PADBLK, 128), lambda i: (i, 0)),
        out_shape=jax.ShapeDtypeStruct((TPAD_ROWS, 128), jnp.bfloat16),
    )(tT, eye)


@functools.lru_cache(maxsize=None)
def _make_sc_gather(B):
    info = plsc.get_sparse_core_info()
    NC, NS = info.num_cores, info.num_subcores
    NW = NC * NS
    bw = B // NW
    mesh = plsc.VectorSubcoreMesh(core_axis_name="c", subcore_axis_name="s")

    @functools.partial(
        pl.kernel,
        mesh=mesh,
        out_type=jax.ShapeDtypeStruct((B, 128), jnp.bfloat16),
        scratch_types=[
            pltpu.VMEM((bw,), jnp.int32),
            pltpu.VMEM((bw, 128), jnp.bfloat16),
            pltpu.SemaphoreType.DMA,
        ],
        compiler_params=pltpu.CompilerParams(use_tc_tiling_on_sc=False),
    )
    def gather_kernel(tp_hbm, idx_hbm, out_hbm, q_v, rows_v, sem):
        wid = lax.axis_index("s") * NC + lax.axis_index("c")
        base = wid * bw
        pltpu.sync_copy(idx_hbm.at[pl.ds(base, bw)], q_v)
        pltpu.async_copy(tp_hbm.at[q_v], rows_v, sem).wait()
        pltpu.sync_copy(rows_v, out_hbm.at[pl.ds(base, bw)])

    return gather_kernel


def _mlp_body(x_ref, w1_ref, b1_ref, w2_ref, b2_ref, w3_ref, b3_ref, o_ref):
    x = x_ref[...].astype(jnp.float32)
    h = jnp.dot(x, w1_ref[...], preferred_element_type=jnp.float32)
    h = jnp.maximum(h + b1_ref[...], 0.0)
    h = jnp.dot(h, w2_ref[...], preferred_element_type=jnp.float32)
    h = jnp.maximum(h + b2_ref[...], 0.0)
    o = jnp.dot(h, w3_ref[...], preferred_element_type=jnp.float32)
    o_ref[...] = o + b3_ref[...]


def _tc_mlp(xp, W1, b1, W2, b2, W3, b3):
    w1p = jnp.pad(W1, ((0, 128 - EMBED_DIM), (0, 0)))
    BB = 2048
    grid = (BATCH // BB,)
    full = lambda i: (0, 0)
    return pl.pallas_call(
        _mlp_body,
        grid=grid,
        in_specs=[
            pl.BlockSpec((BB, 128), lambda i: (i, 0)),
            pl.BlockSpec((128, H1), full),
            pl.BlockSpec((1, H1), full),
            pl.BlockSpec((H1, H2), full),
            pl.BlockSpec((1, H2), full),
            pl.BlockSpec((H2, H3), full),
            pl.BlockSpec((1, H3), full),
        ],
        out_specs=pl.BlockSpec((BB, H3), lambda i: (i, 0)),
        out_shape=jax.ShapeDtypeStruct((BATCH, H3), jnp.float32),
    )(xp, w1p, b1.reshape(1, H1), W2, b2.reshape(1, H2), W3,
      b3.reshape(1, H3))


def kernel(ids, table, W1, b1, W2, b2, W3, b3):
    tp = _tc_pad(table.T)
    xp = _make_sc_gather(BATCH)(tp, ids.astype(jnp.int32))
    return _tc_mlp(xp, W1, b1, W2, b2, W3, b3)


# speedup vs baseline: 2.3480x; 2.3480x over previous
import functools

import jax
import jax.numpy as jnp
from jax import lax
from jax.experimental import pallas as pl
from jax.experimental.pallas import tpu as pltpu
from jax.experimental.pallas import tpu_sc as plsc

VOCAB = 100000
EMBED_DIM = 32
BATCH = 16384
H1, H2, H3 = 128, 64, 32
TPAD_ROWS = 100352
QUADS = TPAD_ROWS // 4


_PADBLK = 2048


def _padt_body(xt_ref, e_ref, o_ref):
    e = e_ref[...]
    for t in range(_PADBLK // 128):
        tile = xt_ref[:, pl.ds(t * 128, 128)]
        tt = lax.dot_general(
            e, tile, (((1,), (1,)), ((), ())),
            preferred_element_type=jnp.float32,
        )
        o_ref[pl.ds((t // 4) * 128, 128), pl.ds((t % 4) * 32, 32)] = tt


def _tc_pad(tT):
    nblk = (TPAD_ROWS + _PADBLK - 1) // _PADBLK
    eye = jnp.eye(128, dtype=jnp.float32)
    return pl.pallas_call(
        _padt_body,
        grid=(nblk,),
        in_specs=[
            pl.BlockSpec((EMBED_DIM, _PADBLK), lambda i: (0, i)),
            pl.BlockSpec((128, 128), lambda i: (0, 0)),
        ],
        out_specs=pl.BlockSpec((_PADBLK // 4, 128), lambda i: (i, 0)),
        out_shape=jax.ShapeDtypeStruct((QUADS, 128), jnp.float32),
    )(tT, eye)


@functools.lru_cache(maxsize=None)
def _make_sc_gather(B, D):
    info = plsc.get_sparse_core_info()
    NC, NS = info.num_cores, info.num_subcores
    NW = NC * NS
    bw = B // NW
    mesh = plsc.VectorSubcoreMesh(core_axis_name="c", subcore_axis_name="s")

    @functools.partial(
        pl.kernel,
        mesh=mesh,
        out_type=jax.ShapeDtypeStruct((B, D), jnp.float32),
        scratch_types=[
            pltpu.VMEM((bw,), jnp.int32),
            pltpu.VMEM((bw, D), jnp.float32),
            pltpu.SemaphoreType.DMA,
        ],
        compiler_params=pltpu.CompilerParams(use_tc_tiling_on_sc=False),
    )
    def gather_kernel(view_hbm, idx_hbm, out_hbm, q_v, rows_v, sem):
        wid = lax.axis_index("s") * NC + lax.axis_index("c")
        base = wid * bw
        pltpu.sync_copy(idx_hbm.at[pl.ds(base, bw)], q_v)
        L = 16
        for k in range(bw // L):
            sl = pl.ds(k * L, L)
            r = q_v[sl]
            q_v[sl] = (
                lax.shift_left(lax.shift_right_logical(r, 9), 9)
                | lax.shift_left(jnp.bitwise_and(r, 127), 2)
                | jnp.bitwise_and(lax.shift_right_logical(r, 7), 3)
            )
        pltpu.async_copy(view_hbm.at[q_v], rows_v, sem).wait()
        pltpu.sync_copy(rows_v, out_hbm.at[pl.ds(base, bw)])

    return gather_kernel


def _mlp_body(x_ref, w1_ref, b1_ref, w2_ref, b2_ref, w3_ref, b3_ref, o_ref):
    x = x_ref[...]
    h = jnp.dot(x, w1_ref[...], preferred_element_type=jnp.float32)
    h = jnp.maximum(h + b1_ref[...], 0.0)
    h = jnp.dot(h, w2_ref[...], preferred_element_type=jnp.float32)
    h = jnp.maximum(h + b2_ref[...], 0.0)
    o = jnp.dot(h, w3_ref[...], preferred_element_type=jnp.float32)
    o_ref[...] = o + b3_ref[...]


def _tc_mlp_quad(x4, W1, b1, W2, b2, W3, b3):
    eye4 = jnp.eye(4, dtype=jnp.float32)
    w1q = jnp.kron(eye4, W1)
    w2q = jnp.kron(eye4, W2)
    w3q = jnp.kron(eye4, W3)
    b1q = jnp.tile(b1, 4).reshape(1, 4 * H1)
    b2q = jnp.tile(b2, 4).reshape(1, 4 * H2)
    b3q = jnp.tile(b3, 4).reshape(1, 4 * H3)
    BQ = BATCH // 4
    BB = 512
    grid = (BQ // BB,)
    full = lambda i: (0, 0)
    return pl.pallas_call(
        _mlp_body,
        grid=grid,
        in_specs=[
            pl.BlockSpec((BB, 128), lambda i: (i, 0)),
            pl.BlockSpec((128, 4 * H1), full),
            pl.BlockSpec((1, 4 * H1), full),
            pl.BlockSpec((4 * H1, 4 * H2), full),
            pl.BlockSpec((1, 4 * H2), full),
            pl.BlockSpec((4 * H2, 4 * H3), full),
            pl.BlockSpec((1, 4 * H3), full),
        ],
        out_specs=pl.BlockSpec((BB, 128), lambda i: (i, 0)),
        out_shape=jax.ShapeDtypeStruct((BQ, 128), jnp.float32),
    )(x4, w1q, b1q, w2q, b2q, w3q, b3q)


def kernel(ids, table, W1, b1, W2, b2, W3, b3):
    t4 = _tc_pad(table.T)
    view = t4.reshape(TPAD_ROWS, EMBED_DIM)
    x = _make_sc_gather(BATCH, EMBED_DIM)(view, ids.astype(jnp.int32))
    x4 = x.reshape(BATCH // 4, 128)
    o4 = _tc_mlp_quad(x4, W1, b1, W2, b2, W3, b3)
    return o4.reshape(BATCH, H3)


# speedup vs baseline: 2.8813x; 1.2272x over previous
import functools

import jax
import jax.numpy as jnp
from jax import lax
from jax.experimental import pallas as pl
from jax.experimental.pallas import tpu as pltpu
from jax.experimental.pallas import tpu_sc as plsc

VOCAB = 100000
EMBED_DIM = 32
BATCH = 16384
H1, H2, H3 = 128, 64, 32
TPAD_ROWS = 100352
QUADS = TPAD_ROWS // 4


_PADBLK = 4096


def _padt_body(xt_ref, e_ref, o_ref):
    e = e_ref[...]
    for t in range(_PADBLK // 128):
        tile = xt_ref[:, pl.ds(t * 128, 128)]
        tt = lax.dot_general(
            e, tile, (((1,), (1,)), ((), ())),
            preferred_element_type=jnp.float32,
        )
        o_ref[pl.ds((t // 4) * 128, 128), pl.ds((t % 4) * 32, 32)] = tt


def _tc_pad(tT):
    nblk = (TPAD_ROWS + _PADBLK - 1) // _PADBLK
    eye = jnp.eye(128, dtype=jnp.float32)
    return pl.pallas_call(
        _padt_body,
        grid=(nblk,),
        in_specs=[
            pl.BlockSpec((EMBED_DIM, _PADBLK), lambda i: (0, i)),
            pl.BlockSpec((128, 128), lambda i: (0, 0)),
        ],
        out_specs=pl.BlockSpec((_PADBLK // 4, 128), lambda i: (i, 0)),
        out_shape=jax.ShapeDtypeStruct((QUADS, 128), jnp.float32),
    )(tT, eye)


@functools.lru_cache(maxsize=None)
def _make_sc_gather(B, D):
    info = plsc.get_sparse_core_info()
    NC, NS = info.num_cores, info.num_subcores
    NW = NC * NS
    bw = B // NW
    mesh = plsc.VectorSubcoreMesh(core_axis_name="c", subcore_axis_name="s")

    @functools.partial(
        pl.kernel,
        mesh=mesh,
        out_type=jax.ShapeDtypeStruct((B, D), jnp.float32),
        scratch_types=[
            pltpu.VMEM((bw,), jnp.int32),
            pltpu.VMEM((bw, D), jnp.float32),
            pltpu.SemaphoreType.DMA,
        ],
        compiler_params=pltpu.CompilerParams(use_tc_tiling_on_sc=False),
    )
    def gather_kernel(view_hbm, idx_hbm, out_hbm, q_v, rows_v, sem):
        wid = lax.axis_index("s") * NC + lax.axis_index("c")
        base = wid * bw
        pltpu.sync_copy(idx_hbm.at[pl.ds(base, bw)], q_v)
        L = 16
        for k in range(bw // L):
            sl = pl.ds(k * L, L)
            r = q_v[sl]
            q_v[sl] = (
                lax.shift_left(lax.shift_right_logical(r, 9), 9)
                | lax.shift_left(jnp.bitwise_and(r, 127), 2)
                | jnp.bitwise_and(lax.shift_right_logical(r, 7), 3)
            )
        pltpu.async_copy(view_hbm.at[q_v], rows_v, sem).wait()
        pltpu.sync_copy(rows_v, out_hbm.at[pl.ds(base, bw)])

    return gather_kernel


def _mlp_body(x_ref, w1_ref, b1_ref, w2_ref, b2_ref, w3_ref, b3_ref, o_ref):
    x = x_ref[...]
    h = jnp.dot(x, w1_ref[...], preferred_element_type=jnp.float32)
    h = jnp.maximum(h + b1_ref[...], 0.0)
    h = jnp.dot(h, w2_ref[...], preferred_element_type=jnp.float32)
    h = jnp.maximum(h + b2_ref[...], 0.0)
    o = jnp.dot(h, w3_ref[...], preferred_element_type=jnp.float32)
    o_ref[...] = o + b3_ref[...]


def _tc_mlp_quad(x4, W1, b1, W2, b2, W3, b3):
    eye4 = jnp.eye(4, dtype=jnp.float32)
    w1q = jnp.kron(eye4, W1)
    w2q = jnp.kron(eye4, W2)
    w3q = jnp.kron(eye4, W3)
    b1q = jnp.tile(b1, 4).reshape(1, 4 * H1)
    b2q = jnp.tile(b2, 4).reshape(1, 4 * H2)
    b3q = jnp.tile(b3, 4).reshape(1, 4 * H3)
    BQ = BATCH // 4
    BB = 1024
    grid = (BQ // BB,)
    full = lambda i: (0, 0)
    return pl.pallas_call(
        _mlp_body,
        grid=grid,
        in_specs=[
            pl.BlockSpec((BB, 128), lambda i: (i, 0)),
            pl.BlockSpec((128, 4 * H1), full),
            pl.BlockSpec((1, 4 * H1), full),
            pl.BlockSpec((4 * H1, 4 * H2), full),
            pl.BlockSpec((1, 4 * H2), full),
            pl.BlockSpec((4 * H2, 4 * H3), full),
            pl.BlockSpec((1, 4 * H3), full),
        ],
        out_specs=pl.BlockSpec((BB, 128), lambda i: (i, 0)),
        out_shape=jax.ShapeDtypeStruct((BQ, 128), jnp.float32),
    )(x4, w1q, b1q, w2q, b2q, w3q, b3q)


def kernel(ids, table, W1, b1, W2, b2, W3, b3):
    t4 = _tc_pad(table.T)
    view = t4.reshape(TPAD_ROWS, EMBED_DIM)
    x = _make_sc_gather(BATCH, EMBED_DIM)(view, ids.astype(jnp.int32))
    x4 = x.reshape(BATCH // 4, 128)
    o4 = _tc_mlp_quad(x4, W1, b1, W2, b2, W3, b3)
    return o4.reshape(BATCH, H3)


# speedup vs baseline: 3.1853x; 1.1055x over previous
import functools

import jax
import jax.numpy as jnp
from jax import lax
from jax.experimental import pallas as pl
from jax.experimental.pallas import tpu as pltpu
from jax.experimental.pallas import tpu_sc as plsc

VOCAB = 100000
EMBED_DIM = 32
BATCH = 16384
H1, H2, H3 = 128, 64, 32
TPAD_ROWS = 100352
QUADS = TPAD_ROWS // 4


_PADBLK = 8192


def _padt_body(xt_ref, e_ref, o_ref):
    e = e_ref[...]
    for t in range(_PADBLK // 128):
        tile = xt_ref[:, pl.ds(t * 128, 128)]
        tt = lax.dot_general(
            e, tile, (((1,), (1,)), ((), ())),
            preferred_element_type=jnp.float32,
        )
        o_ref[pl.ds((t // 4) * 128, 128), pl.ds((t % 4) * 32, 32)] = tt


def _tc_pad(tT):
    nblk = (TPAD_ROWS + _PADBLK - 1) // _PADBLK
    eye = jnp.eye(128, dtype=jnp.float32)
    return pl.pallas_call(
        _padt_body,
        grid=(nblk,),
        in_specs=[
            pl.BlockSpec((EMBED_DIM, _PADBLK), lambda i: (0, i)),
            pl.BlockSpec((128, 128), lambda i: (0, 0)),
        ],
        out_specs=pl.BlockSpec((_PADBLK // 4, 128), lambda i: (i, 0)),
        out_shape=jax.ShapeDtypeStruct((QUADS, 128), jnp.float32),
    )(tT, eye)


@functools.lru_cache(maxsize=None)
def _make_sc_gather(B, D):
    info = plsc.get_sparse_core_info()
    NC, NS = info.num_cores, info.num_subcores
    NW = NC * NS
    bw = B // NW
    mesh = plsc.VectorSubcoreMesh(core_axis_name="c", subcore_axis_name="s")

    @functools.partial(
        pl.kernel,
        mesh=mesh,
        out_type=jax.ShapeDtypeStruct((B, D), jnp.float32),
        scratch_types=[
            pltpu.VMEM((bw,), jnp.int32),
            pltpu.VMEM((bw, D), jnp.float32),
            pltpu.SemaphoreType.DMA,
        ],
        compiler_params=pltpu.CompilerParams(use_tc_tiling_on_sc=False),
    )
    def gather_kernel(view_hbm, idx_hbm, out_hbm, q_v, rows_v, sem):
        wid = lax.axis_index("s") * NC + lax.axis_index("c")
        base = wid * bw
        pltpu.sync_copy(idx_hbm.at[pl.ds(base, bw)], q_v)
        L = 16
        for k in range(bw // L):
            sl = pl.ds(k * L, L)
            r = q_v[sl]
            q_v[sl] = (
                lax.shift_left(lax.shift_right_logical(r, 9), 9)
                | lax.shift_left(jnp.bitwise_and(r, 127), 2)
                | jnp.bitwise_and(lax.shift_right_logical(r, 7), 3)
            )
        pltpu.async_copy(view_hbm.at[q_v], rows_v, sem).wait()
        pltpu.sync_copy(rows_v, out_hbm.at[pl.ds(base, bw)])

    return gather_kernel


def _mlp_body(x_ref, w1_ref, b1_ref, w2_ref, b2_ref, w3_ref, b3_ref, o_ref):
    x = x_ref[...]
    h = jnp.dot(x, w1_ref[...], preferred_element_type=jnp.float32)
    h = jnp.maximum(h + b1_ref[...], 0.0)
    h = jnp.dot(h, w2_ref[...], preferred_element_type=jnp.float32)
    h = jnp.maximum(h + b2_ref[...], 0.0)
    o = jnp.dot(h, w3_ref[...], preferred_element_type=jnp.float32)
    o_ref[...] = o + b3_ref[...]


def _tc_mlp_quad(x4, W1, b1, W2, b2, W3, b3):
    eye4 = jnp.eye(4, dtype=jnp.float32)
    w1q = jnp.kron(eye4, W1)
    w2q = jnp.kron(eye4, W2)
    w3q = jnp.kron(eye4, W3)
    b1q = jnp.tile(b1, 4).reshape(1, 4 * H1)
    b2q = jnp.tile(b2, 4).reshape(1, 4 * H2)
    b3q = jnp.tile(b3, 4).reshape(1, 4 * H3)
    BQ = BATCH // 4
    BB = 1024
    grid = (BQ // BB,)
    full = lambda i: (0, 0)
    return pl.pallas_call(
        _mlp_body,
        grid=grid,
        in_specs=[
            pl.BlockSpec((BB, 128), lambda i: (i, 0)),
            pl.BlockSpec((128, 4 * H1), full),
            pl.BlockSpec((1, 4 * H1), full),
            pl.BlockSpec((4 * H1, 4 * H2), full),
            pl.BlockSpec((1, 4 * H2), full),
            pl.BlockSpec((4 * H2, 4 * H3), full),
            pl.BlockSpec((1, 4 * H3), full),
        ],
        out_specs=pl.BlockSpec((BB, 128), lambda i: (i, 0)),
        out_shape=jax.ShapeDtypeStruct((BQ, 128), jnp.float32),
    )(x4, w1q, b1q, w2q, b2q, w3q, b3q)


def kernel(ids, table, W1, b1, W2, b2, W3, b3):
    t4 = _tc_pad(table.T)
    view = t4.reshape(TPAD_ROWS, EMBED_DIM)
    x = _make_sc_gather(BATCH, EMBED_DIM)(view, ids.astype(jnp.int32))
    x4 = x.reshape(BATCH // 4, 128)
    o4 = _tc_mlp_quad(x4, W1, b1, W2, b2, W3, b3)
    return o4.reshape(BATCH, H3)


# speedup vs baseline: 3.3347x; 1.0469x over previous
import functools

import jax
import jax.numpy as jnp
from jax import lax
from jax.experimental import pallas as pl
from jax.experimental.pallas import tpu as pltpu
from jax.experimental.pallas import tpu_sc as plsc

VOCAB = 100000
EMBED_DIM = 32
BATCH = 16384
H1, H2, H3 = 128, 64, 32
TPAD_ROWS = 100352
QUADS = TPAD_ROWS // 4


_PADBLK = 16384


def _padt_body(xt_ref, e_ref, o_ref):
    e = e_ref[...]
    for t in range(_PADBLK // 128):
        tile = xt_ref[:, pl.ds(t * 128, 128)]
        tt = lax.dot_general(
            e, tile, (((1,), (1,)), ((), ())),
            preferred_element_type=jnp.float32,
        )
        o_ref[pl.ds((t // 4) * 128, 128), pl.ds((t % 4) * 32, 32)] = tt


def _tc_pad(tT):
    nblk = (TPAD_ROWS + _PADBLK - 1) // _PADBLK
    eye = jnp.eye(128, dtype=jnp.float32)
    return pl.pallas_call(
        _padt_body,
        grid=(nblk,),
        in_specs=[
            pl.BlockSpec((EMBED_DIM, _PADBLK), lambda i: (0, i)),
            pl.BlockSpec((128, 128), lambda i: (0, 0)),
        ],
        out_specs=pl.BlockSpec((_PADBLK // 4, 128), lambda i: (i, 0)),
        out_shape=jax.ShapeDtypeStruct((QUADS, 128), jnp.float32),
    )(tT, eye)


@functools.lru_cache(maxsize=None)
def _make_sc_gather(B, D):
    info = plsc.get_sparse_core_info()
    NC, NS = info.num_cores, info.num_subcores
    NW = NC * NS
    bw = B // NW
    mesh = plsc.VectorSubcoreMesh(core_axis_name="c", subcore_axis_name="s")

    @functools.partial(
        pl.kernel,
        mesh=mesh,
        out_type=jax.ShapeDtypeStruct((B, D), jnp.float32),
        scratch_types=[
            pltpu.VMEM((bw,), jnp.int32),
            pltpu.VMEM((bw, D), jnp.float32),
            pltpu.SemaphoreType.DMA,
        ],
        compiler_params=pltpu.CompilerParams(use_tc_tiling_on_sc=False),
    )
    def gather_kernel(view_hbm, idx_hbm, out_hbm, q_v, rows_v, sem):
        wid = lax.axis_index("s") * NC + lax.axis_index("c")
        base = wid * bw
        pltpu.sync_copy(idx_hbm.at[pl.ds(base, bw)], q_v)
        L = 16
        for k in range(bw // L):
            sl = pl.ds(k * L, L)
            r = q_v[sl]
            q_v[sl] = (
                lax.shift_left(lax.shift_right_logical(r, 9), 9)
                | lax.shift_left(jnp.bitwise_and(r, 127), 2)
                | jnp.bitwise_and(lax.shift_right_logical(r, 7), 3)
            )
        pltpu.async_copy(view_hbm.at[q_v], rows_v, sem).wait()
        pltpu.sync_copy(rows_v, out_hbm.at[pl.ds(base, bw)])

    return gather_kernel


def _mlp_body(x_ref, w1_ref, b1_ref, w2_ref, b2_ref, w3_ref, b3_ref, o_ref):
    x = x_ref[...]
    h = jnp.dot(x, w1_ref[...], preferred_element_type=jnp.float32)
    h = jnp.maximum(h + b1_ref[...], 0.0)
    h = jnp.dot(h, w2_ref[...], preferred_element_type=jnp.float32)
    h = jnp.maximum(h + b2_ref[...], 0.0)
    o = jnp.dot(h, w3_ref[...], preferred_element_type=jnp.float32)
    o_ref[...] = o + b3_ref[...]


def _tc_mlp_quad(x4, W1, b1, W2, b2, W3, b3):
    eye4 = jnp.eye(4, dtype=jnp.float32)
    w1q = jnp.kron(eye4, W1)
    w2q = jnp.kron(eye4, W2)
    w3q = jnp.kron(eye4, W3)
    b1q = jnp.tile(b1, 4).reshape(1, 4 * H1)
    b2q = jnp.tile(b2, 4).reshape(1, 4 * H2)
    b3q = jnp.tile(b3, 4).reshape(1, 4 * H3)
    BQ = BATCH // 4
    BB = 1024
    grid = (BQ // BB,)
    full = lambda i: (0, 0)
    return pl.pallas_call(
        _mlp_body,
        grid=grid,
        in_specs=[
            pl.BlockSpec((BB, 128), lambda i: (i, 0)),
            pl.BlockSpec((128, 4 * H1), full),
            pl.BlockSpec((1, 4 * H1), full),
            pl.BlockSpec((4 * H1, 4 * H2), full),
            pl.BlockSpec((1, 4 * H2), full),
            pl.BlockSpec((4 * H2, 4 * H3), full),
            pl.BlockSpec((1, 4 * H3), full),
        ],
        out_specs=pl.BlockSpec((BB, 128), lambda i: (i, 0)),
        out_shape=jax.ShapeDtypeStruct((BQ, 128), jnp.float32),
    )(x4, w1q, b1q, w2q, b2q, w3q, b3q)


def kernel(ids, table, W1, b1, W2, b2, W3, b3):
    t4 = _tc_pad(table.T)
    view = t4.reshape(TPAD_ROWS, EMBED_DIM)
    x = _make_sc_gather(BATCH, EMBED_DIM)(view, ids.astype(jnp.int32))
    x4 = x.reshape(BATCH // 4, 128)
    o4 = _tc_mlp_quad(x4, W1, b1, W2, b2, W3, b3)
    return o4.reshape(BATCH, H3)
